# Initial kernel scaffold; baseline (speedup 1.0000x reference)
import jax
import jax.numpy as jnp
from jax.experimental import pallas as pl


def kernel(inputs, ent_emb, rel_emb):
    raise NotImplementedError("write your pallas kernel here")


# speedup vs baseline: 2.5516x; 2.5516x over previous
import functools

import jax
import jax.numpy as jnp
from jax import lax
from jax.experimental import pallas as pl
from jax.experimental.pallas import tpu as pltpu
from jax.experimental.pallas import tpu_sc as plsc

ETA_C = 20
K_C = 128
MAX_ENT_C = 100000
BATCH_C = 4096
NC, NS, L = 2, 16, 16
NW = NC * NS
PW = BATCH_C // NW
NCH = K_C // L
NG = PW // L


def _body(s_idx, p_idx, o_idx, keep, repl, ent, rel, out_inp, out_corr,
          sidx_v, pidx_v, oidx_v, ri_a, ri_b, k_v, w_buf, p_buf, r_a, r_b,
          inp_v, corr_v, sem_a, sem_b, sem_p):
    wid = lax.axis_index("s") * NC + lax.axis_index("c")
    base = wid * PW
    lane = lax.broadcasted_iota(jnp.int32, (L,), 0)

    pltpu.sync_copy(s_idx.at[pl.ds(base, PW)], sidx_v)
    pltpu.sync_copy(o_idx.at[pl.ds(base, PW)], oidx_v)
    pltpu.sync_copy(p_idx.at[pl.ds(base, PW)], pidx_v)
    cs = pltpu.async_copy(ent.at[sidx_v], w_buf.at[pl.ds(0, PW)], sem_a)
    co = pltpu.async_copy(ent.at[oidx_v], w_buf.at[pl.ds(PW, PW)], sem_b)
    cp = pltpu.async_copy(rel.at[pidx_v], p_buf, sem_p)
    cs.wait()
    co.wait()
    cp.wait()

    def orig_group(g, carry):
        del carry
        vec = jnp.zeros((L,), jnp.float32)
        for l in range(L):
            i = g * L + l
            acc = jnp.zeros((L,), jnp.float32)
            for c in range(NCH):
                sl = pl.ds(c * L, L)
                s = w_buf[i, sl]
                o = w_buf[PW + i, sl]
                p = p_buf[i, sl]
                sp = s * p
                po = p * o
                acc = acc + sp * o
                w_buf[i, sl] = sp - po
                w_buf[PW + i, sl] = po
            tot = jnp.sum(acc)
            vec = jnp.where(lane == l, lax.broadcast_in_dim(tot, (L,), ()), vec)
        inp_v[pl.ds(g * L, L)] = vec
        return 0

    lax.fori_loop(0, NG, orig_group, 0)
    pltpu.sync_copy(inp_v, out_inp.at[pl.ds(base, PW)])

    def corr_chunk(tt, r_buf, ri_v, sem):
        off = tt * BATCH_C + base
        pltpu.sync_copy(repl.at[pl.ds(off, PW)], ri_v)
        gat = pltpu.async_copy(ent.at[ri_v], r_buf, sem)
        pltpu.sync_copy(keep.at[pl.ds(off, PW)], k_v)
        gat.wait()

        def group(g, carry):
            del carry
            kf = k_v[pl.ds(g * L, L)].astype(jnp.float32)
            sc_po = jnp.zeros((L,), jnp.float32)
            sc_d = jnp.zeros((L,), jnp.float32)
            for l in range(L):
                i = g * L + l
                accp = jnp.zeros((L,), jnp.float32)
                accd = jnp.zeros((L,), jnp.float32)
                for c in range(NCH):
                    sl = pl.ds(c * L, L)
                    r = r_buf[i, sl]
                    accp = accp + w_buf[PW + i, sl] * r
                    accd = accd + w_buf[i, sl] * r
                tp = jnp.sum(accp)
                td = jnp.sum(accd)
                sc_po = jnp.where(lane == l,
                                  lax.broadcast_in_dim(tp, (L,), ()), sc_po)
                sc_d = jnp.where(lane == l,
                                 lax.broadcast_in_dim(td, (L,), ()), sc_d)
            corr_v[pl.ds(g * L, L)] = sc_po + kf * sc_d
            return 0

        lax.fori_loop(0, NG, group, 0)
        pltpu.sync_copy(corr_v, out_corr.at[pl.ds(off, PW)])

    def t_pair(u, carry):
        del carry
        corr_chunk(2 * u, r_a, ri_a, sem_a)
        corr_chunk(2 * u + 1, r_b, ri_b, sem_b)
        return 0

    lax.fori_loop(0, ETA_C // 2, t_pair, 0)


_sc_call = pl.kernel(
    _body,
    out_type=(
        jax.ShapeDtypeStruct((BATCH_C,), jnp.float32),
        jax.ShapeDtypeStruct((ETA_C * BATCH_C,), jnp.float32),
    ),
    mesh=plsc.VectorSubcoreMesh(core_axis_name="c", subcore_axis_name="s"),
    compiler_params=pltpu.CompilerParams(needs_layout_passes=False),
    scratch_types=[
        pltpu.VMEM((PW,), jnp.int32),
        pltpu.VMEM((PW,), jnp.int32),
        pltpu.VMEM((PW,), jnp.int32),
        pltpu.VMEM((PW,), jnp.int32),
        pltpu.VMEM((PW,), jnp.int32),
        pltpu.VMEM((PW,), jnp.int32),
        pltpu.VMEM((2 * PW, K_C), jnp.float32),
        pltpu.VMEM((PW, K_C), jnp.float32),
        pltpu.VMEM((PW, K_C), jnp.float32),
        pltpu.VMEM((PW, K_C), jnp.float32),
        pltpu.VMEM((PW,), jnp.float32),
        pltpu.VMEM((PW,), jnp.float32),
        pltpu.SemaphoreType.DMA,
        pltpu.SemaphoreType.DMA,
        pltpu.SemaphoreType.DMA,
    ],
)


@jax.jit
def kernel(inputs, ent_emb, rel_emb):
    s_idx = inputs[:, 0]
    p_idx = inputs[:, 1]
    o_idx = inputs[:, 2]
    ckey = jax.random.key(42)
    ka, kb = jax.random.split(ckey)
    n = ETA_C * BATCH_C
    keep = jax.random.randint(ka, (n,), 0, 2, dtype=jnp.int32)
    repl = jax.random.randint(kb, (n,), 0, MAX_ENT_C, dtype=jnp.int32)
    inp_score, corr_score = _sc_call(
        s_idx, p_idx, o_idx, keep, repl, ent_emb, rel_emb)
    return (inp_score, corr_score)


# speedup vs baseline: 3.3320x; 1.3058x over previous
import jax
import jax.numpy as jnp
import numpy as np
from jax import lax
from jax.experimental import pallas as pl
from jax.experimental.pallas import tpu as pltpu
from jax.experimental.pallas import tpu_sc as plsc

ETA_C = 20
K_C = 128
MAX_ENT_C = 100000
BATCH_C = 4096
NC, NS, L = 2, 16, 16
NW = NC * NS
PW = BATCH_C // NW
NCH = K_C // L
NG = PW // L

_N_CORR = ETA_C * BATCH_C


def _body(s_idx, p_idx, o_idx, keep, repl, ent, rel, out_inp, out_corr,
          sidx_v, pidx_v, oidx_v, ri_all, k_all, w_buf, p_buf, r_a, r_b,
          inp_v, corr_all, sem_s, sem_o, sem_p, sem_a, sem_b):
    wid = lax.axis_index("s") * NC + lax.axis_index("c")
    base = wid * PW
    lane = lax.broadcasted_iota(jnp.int32, (L,), 0)

    pltpu.sync_copy(s_idx.at[pl.ds(base, PW)], sidx_v)
    pltpu.sync_copy(o_idx.at[pl.ds(base, PW)], oidx_v)
    pltpu.sync_copy(p_idx.at[pl.ds(base, PW)], pidx_v)
    pltpu.sync_copy(repl.at[:, pl.ds(base, PW)], ri_all)
    pltpu.sync_copy(keep.at[:, pl.ds(base, PW)], k_all)
    cs = pltpu.async_copy(ent.at[sidx_v], w_buf.at[pl.ds(0, PW)], sem_s)
    co = pltpu.async_copy(ent.at[oidx_v], w_buf.at[pl.ds(PW, PW)], sem_o)
    cp = pltpu.async_copy(rel.at[pidx_v], p_buf, sem_p)

    def fire(t, r_buf, sem):
        return pltpu.async_copy(ent.at[ri_all.at[t]], r_buf, sem)

    def gwait(t, r_buf, sem):
        pltpu.make_async_copy(ent.at[ri_all.at[t]], r_buf, sem).wait()

    ga = fire(0, r_a, sem_a)
    gb = fire(1, r_b, sem_b)

    cs.wait()
    co.wait()
    cp.wait()

    def orig_group(g, carry):
        del carry
        vec = jnp.zeros((L,), jnp.float32)
        for l in range(L):
            i = g * L + l
            acc0 = jnp.zeros((L,), jnp.float32)
            acc1 = jnp.zeros((L,), jnp.float32)
            for c in range(NCH):
                sl = pl.ds(c * L, L)
                s = w_buf[i, sl]
                o = w_buf[PW + i, sl]
                p = p_buf[i, sl]
                sp = s * p
                po = p * o
                if c % 2 == 0:
                    acc0 = acc0 + sp * o
                else:
                    acc1 = acc1 + sp * o
                w_buf[i, sl] = sp - po
                w_buf[PW + i, sl] = po
            tot = jnp.sum(acc0 + acc1)
            vec = jnp.where(lane == l, lax.broadcast_in_dim(tot, (L,), ()), vec)
        inp_v[pl.ds(g * L, L)] = vec
        return 0

    lax.fori_loop(0, NG, orig_group, 0)
    pltpu.sync_copy(inp_v, out_inp.at[pl.ds(base, PW)])

    def score_chunk(tt, r_buf):
        def group(g, carry):
            del carry
            kf = k_all[tt, pl.ds(g * L, L)].astype(jnp.float32)
            sc_po = jnp.zeros((L,), jnp.float32)
            sc_d = jnp.zeros((L,), jnp.float32)
            for l in range(L):
                i = g * L + l
                ap0 = jnp.zeros((L,), jnp.float32)
                ap1 = jnp.zeros((L,), jnp.float32)
                ad0 = jnp.zeros((L,), jnp.float32)
                ad1 = jnp.zeros((L,), jnp.float32)
                for c in range(NCH):
                    sl = pl.ds(c * L, L)
                    r = r_buf[i, sl]
                    if c % 2 == 0:
                        ap0 = ap0 + w_buf[PW + i, sl] * r
                        ad0 = ad0 + w_buf[i, sl] * r
                    else:
                        ap1 = ap1 + w_buf[PW + i, sl] * r
                        ad1 = ad1 + w_buf[i, sl] * r
                tp = jnp.sum(ap0 + ap1)
                td = jnp.sum(ad0 + ad1)
                sc_po = jnp.where(lane == l,
                                  lax.broadcast_in_dim(tp, (L,), ()), sc_po)
                sc_d = jnp.where(lane == l,
                                 lax.broadcast_in_dim(td, (L,), ()), sc_d)
            corr_all[tt, pl.ds(g * L, L)] = sc_po + kf * sc_d
            return 0

        lax.fori_loop(0, NG, group, 0)

    def t_pair(u, carry):
        del carry
        gwait(2 * u, r_a, sem_a)
        score_chunk(2 * u, r_a)
        fire(2 * u + 2, r_a, sem_a)
        gwait(2 * u + 1, r_b, sem_b)
        score_chunk(2 * u + 1, r_b)
        fire(2 * u + 3, r_b, sem_b)
        return 0

    del ga, gb
    lax.fori_loop(0, ETA_C // 2 - 1, t_pair, 0)
    gwait(ETA_C - 2, r_a, sem_a)
    score_chunk(ETA_C - 2, r_a)
    gwait(ETA_C - 1, r_b, sem_b)
    score_chunk(ETA_C - 1, r_b)

    pltpu.sync_copy(corr_all, out_corr.at[:, pl.ds(base, PW)])


_sc_call = pl.kernel(
    _body,
    out_type=(
        jax.ShapeDtypeStruct((BATCH_C,), jnp.float32),
        jax.ShapeDtypeStruct((ETA_C, BATCH_C), jnp.float32),
    ),
    mesh=plsc.VectorSubcoreMesh(core_axis_name="c", subcore_axis_name="s"),
    compiler_params=pltpu.CompilerParams(needs_layout_passes=False),
    scratch_types=[
        pltpu.VMEM((PW,), jnp.int32),
        pltpu.VMEM((PW,), jnp.int32),
        pltpu.VMEM((PW,), jnp.int32),
        pltpu.VMEM((ETA_C, PW), jnp.int32),
        pltpu.VMEM((ETA_C, PW), jnp.int32),
        pltpu.VMEM((2 * PW, K_C), jnp.float32),
        pltpu.VMEM((PW, K_C), jnp.float32),
        pltpu.VMEM((PW, K_C), jnp.float32),
        pltpu.VMEM((PW, K_C), jnp.float32),
        pltpu.VMEM((PW,), jnp.float32),
        pltpu.VMEM((ETA_C, PW), jnp.float32),
        pltpu.SemaphoreType.DMA,
        pltpu.SemaphoreType.DMA,
        pltpu.SemaphoreType.DMA,
        pltpu.SemaphoreType.DMA,
        pltpu.SemaphoreType.DMA,
    ],
)


@jax.jit
def kernel(inputs, ent_emb, rel_emb):
    s_idx = inputs[:, 0]
    p_idx = inputs[:, 1]
    o_idx = inputs[:, 2]
    ckey = jax.random.key(42)
    ka, kb = jax.random.split(ckey)
    keep = jax.random.randint(
        ka, (_N_CORR,), 0, 2, dtype=jnp.int32).reshape(ETA_C, BATCH_C)
    repl = jax.random.randint(
        kb, (_N_CORR,), 0, MAX_ENT_C, dtype=jnp.int32).reshape(ETA_C, BATCH_C)
    inp_score, corr2 = _sc_call(
        s_idx, p_idx, o_idx, keep, repl, ent_emb, rel_emb)
    return (inp_score, corr2.reshape(ETA_C * BATCH_C))


# speedup vs baseline: 4.2879x; 1.2869x over previous
import jax
import jax.numpy as jnp
import numpy as np
from jax import lax
from jax.experimental import pallas as pl
from jax.experimental.pallas import tpu as pltpu
from jax.experimental.pallas import tpu_sc as plsc

ETA_C = 20
K_C = 128
MAX_ENT_C = 100000
BATCH_C = 4096
NC, NS, L = 2, 16, 16
NW = NC * NS
PW = BATCH_C // NW
NCH = K_C // L
NG = PW // L
TS = 17
_N_CORR = ETA_C * BATCH_C


def _row_tree_sum(tile):
    rows = [tile[pl.ds(l * TS, L)] for l in range(L)]
    while len(rows) > 1:
        rows = [rows[k] + rows[k + 1] for k in range(0, len(rows), 2)]
    return rows[0]


def _body(s_idx, p_idx, o_idx, keep, repl, ent, rel, out_inp, out_corr,
          sidx_v, pidx_v, oidx_v, ri_all, k_all, w_buf, p_buf,
          r_a, r_b, r_c, r_d, inp_v, corr_all, t_p0, t_d0, t_p1, t_d1,
          sem_s, sem_o, sem_p, sem_a, sem_b, sem_c, sem_d):
    wid = lax.axis_index("s") * NC + lax.axis_index("c")
    base = wid * PW
    lane = lax.broadcasted_iota(jnp.int32, (L,), 0)
    col0 = lane * TS

    pltpu.sync_copy(s_idx.at[pl.ds(base, PW)], sidx_v)
    pltpu.sync_copy(o_idx.at[pl.ds(base, PW)], oidx_v)
    pltpu.sync_copy(p_idx.at[pl.ds(base, PW)], pidx_v)
    pltpu.sync_copy(repl.at[:, pl.ds(base, PW)], ri_all)
    pltpu.sync_copy(keep.at[:, pl.ds(base, PW)], k_all)
    cs = pltpu.async_copy(ent.at[sidx_v], w_buf.at[pl.ds(0, PW)], sem_s)
    co = pltpu.async_copy(ent.at[oidx_v], w_buf.at[pl.ds(PW, PW)], sem_o)
    cp = pltpu.async_copy(rel.at[pidx_v], p_buf, sem_p)

    def fire(t, r_buf, sem):
        pltpu.async_copy(ent.at[ri_all.at[t]], r_buf, sem)

    def gwait(t, r_buf, sem):
        pltpu.make_async_copy(ent.at[ri_all.at[t]], r_buf, sem).wait()

    fire(0, r_a, sem_a)
    fire(1, r_b, sem_b)
    fire(2, r_c, sem_c)
    fire(3, r_d, sem_d)

    cs.wait()
    co.wait()
    cp.wait()

    def orig_group(g, carry):
        del carry
        for l in range(L):
            i = g * L + l
            acc0 = jnp.zeros((L,), jnp.float32)
            acc1 = jnp.zeros((L,), jnp.float32)
            for c in range(NCH):
                sl = pl.ds(c * L, L)
                s = w_buf[i, sl]
                o = w_buf[PW + i, sl]
                p = p_buf[i, sl]
                sp = s * p
                po = p * o
                if c % 2 == 0:
                    acc0 = acc0 + sp * o
                else:
                    acc1 = acc1 + sp * o
                w_buf[i, sl] = sp - po
                w_buf[PW + i, sl] = po
            plsc.store_scatter(t_p0, [col0 + l], acc0 + acc1)
        inp_v[pl.ds(g * L, L)] = _row_tree_sum(t_p0)
        return 0

    lax.fori_loop(0, NG, orig_group, 0)
    pltpu.sync_copy(inp_v, out_inp.at[pl.ds(base, PW)])

    def score_pair(t0, rx, ry):
        t1 = t0 + 1

        def group(g, carry):
            del carry
            for l in range(L):
                i = g * L + l
                ap0 = jnp.zeros((L,), jnp.float32)
                ad0 = jnp.zeros((L,), jnp.float32)
                ap1 = jnp.zeros((L,), jnp.float32)
                ad1 = jnp.zeros((L,), jnp.float32)
                for c in range(NCH):
                    sl = pl.ds(c * L, L)
                    po = w_buf[PW + i, sl]
                    d = w_buf[i, sl]
                    r0 = rx[i, sl]
                    r1 = ry[i, sl]
                    ap0 = ap0 + po * r0
                    ad0 = ad0 + d * r0
                    ap1 = ap1 + po * r1
                    ad1 = ad1 + d * r1
                col = col0 + l
                plsc.store_scatter(t_p0, [col], ap0)
                plsc.store_scatter(t_d0, [col], ad0)
                plsc.store_scatter(t_p1, [col], ap1)
                plsc.store_scatter(t_d1, [col], ad1)
            gl = pl.ds(g * L, L)
            kf0 = k_all[t0, gl].astype(jnp.float32)
            kf1 = k_all[t1, gl].astype(jnp.float32)
            corr_all[t0, gl] = _row_tree_sum(t_p0) + kf0 * _row_tree_sum(t_d0)
            corr_all[t1, gl] = _row_tree_sum(t_p1) + kf1 * _row_tree_sum(t_d1)
            return 0

        lax.fori_loop(0, NG, group, 0)

    def quad(v, carry):
        del carry
        t = 4 * v
        gwait(t, r_a, sem_a)
        gwait(t + 1, r_b, sem_b)
        score_pair(t, r_a, r_b)
        fire(t + 4, r_a, sem_a)
        fire(t + 5, r_b, sem_b)
        gwait(t + 2, r_c, sem_c)
        gwait(t + 3, r_d, sem_d)
        score_pair(t + 2, r_c, r_d)
        fire(t + 6, r_c, sem_c)
        fire(t + 7, r_d, sem_d)
        return 0

    lax.fori_loop(0, ETA_C // 4 - 1, quad, 0)
    gwait(ETA_C - 4, r_a, sem_a)
    gwait(ETA_C - 3, r_b, sem_b)
    score_pair(ETA_C - 4, r_a, r_b)
    gwait(ETA_C - 2, r_c, sem_c)
    gwait(ETA_C - 1, r_d, sem_d)
    score_pair(ETA_C - 2, r_c, r_d)

    pltpu.sync_copy(corr_all, out_corr.at[:, pl.ds(base, PW)])


_sc_call = pl.kernel(
    _body,
    out_type=(
        jax.ShapeDtypeStruct((BATCH_C,), jnp.float32),
        jax.ShapeDtypeStruct((ETA_C, BATCH_C), jnp.float32),
    ),
    mesh=plsc.VectorSubcoreMesh(core_axis_name="c", subcore_axis_name="s"),
    compiler_params=pltpu.CompilerParams(needs_layout_passes=False),
    scratch_types=[
        pltpu.VMEM((PW,), jnp.int32),
        pltpu.VMEM((PW,), jnp.int32),
        pltpu.VMEM((PW,), jnp.int32),
        pltpu.VMEM((ETA_C, PW), jnp.int32),
        pltpu.VMEM((ETA_C, PW), jnp.int32),
        pltpu.VMEM((2 * PW, K_C), jnp.float32),
        pltpu.VMEM((PW, K_C), jnp.float32),
        pltpu.VMEM((PW, K_C), jnp.float32),
        pltpu.VMEM((PW, K_C), jnp.float32),
        pltpu.VMEM((PW, K_C), jnp.float32),
        pltpu.VMEM((PW, K_C), jnp.float32),
        pltpu.VMEM((PW,), jnp.float32),
        pltpu.VMEM((ETA_C, PW), jnp.float32),
        pltpu.VMEM((L * TS,), jnp.float32),
        pltpu.VMEM((L * TS,), jnp.float32),
        pltpu.VMEM((L * TS,), jnp.float32),
        pltpu.VMEM((L * TS,), jnp.float32),
        pltpu.SemaphoreType.DMA,
        pltpu.SemaphoreType.DMA,
        pltpu.SemaphoreType.DMA,
        pltpu.SemaphoreType.DMA,
        pltpu.SemaphoreType.DMA,
        pltpu.SemaphoreType.DMA,
        pltpu.SemaphoreType.DMA,
    ],
)


@jax.jit
def kernel(inputs, ent_emb, rel_emb):
    s_idx = inputs[:, 0]
    p_idx = inputs[:, 1]
    o_idx = inputs[:, 2]
    ckey = jax.random.key(42)
    ka, kb = jax.random.split(ckey)
    keep = jax.random.randint(
        ka, (_N_CORR,), 0, 2, dtype=jnp.int32).reshape(ETA_C, BATCH_C)
    repl = jax.random.randint(
        kb, (_N_CORR,), 0, MAX_ENT_C, dtype=jnp.int32).reshape(ETA_C, BATCH_C)
    inp_score, corr2 = _sc_call(
        s_idx, p_idx, o_idx, keep, repl, ent_emb, rel_emb)
    return (inp_score, corr2.reshape(_N_CORR))
